# Initial kernel scaffold; baseline (speedup 1.0000x reference)
import jax
import jax.numpy as jnp
from jax.experimental import pallas as pl


def kernel(c, delta):
    raise NotImplementedError("write your pallas kernel here")


# speedup vs baseline: 3.9299x; 3.9299x over previous
import jax
import jax.numpy as jnp
from jax import lax
from jax.experimental import pallas as pl
from jax.experimental.pallas import tpu as pltpu

_TI = 0
_TEST = 0.0

_N = 32768
_D = 256
_BLK = 1024


def _body(c_ref, d_ref, o_ref):
    cb = c_ref[...]
    db = d_ref[...]
    lo = cb - db
    hi = cb + db
    left = lo <= _TEST
    right = hi > _TEST
    l_hi = jnp.minimum(hi, _TEST)
    l_c = (lo + l_hi) * 0.5
    l_d = (l_hi - lo) * 0.5
    r_lo = jnp.maximum(lo, _TEST)
    r_c = (r_lo + hi) * 0.5
    r_d = (hi - r_lo) * 0.5
    lo_l = l_c - l_d
    hi_l = l_c + l_d
    lo_r = r_c - r_d
    hi_r = r_c + r_d
    both = left & right
    new_lo = jnp.where(both, jnp.minimum(lo_l, lo_r), jnp.where(left, lo_l, lo_r))
    new_hi = jnp.where(both, jnp.maximum(hi_l, hi_r), jnp.where(left, hi_l, hi_r))
    nc = (new_lo + new_hi) * 0.5
    nd = (new_hi - new_lo) * 0.5
    is_t = lax.broadcasted_iota(jnp.int32, cb.shape, 1) == _TI
    o_ref[0] = jnp.where(is_t, nc, cb)
    o_ref[1] = jnp.where(is_t, nd, db)


def kernel(c, delta):
    grid = (_N // _BLK,)
    return pl.pallas_call(
        _body,
        grid=grid,
        in_specs=[
            pl.BlockSpec((_BLK, _D), lambda i: (i, 0)),
            pl.BlockSpec((_BLK, _D), lambda i: (i, 0)),
        ],
        out_specs=pl.BlockSpec((2, _BLK, _D), lambda i: (0, i, 0)),
        out_shape=jax.ShapeDtypeStruct((2, _N, _D), jnp.float32),
        compiler_params=pltpu.CompilerParams(
            dimension_semantics=("parallel",),
        ),
    )(c, delta)
